# Initial kernel scaffold; baseline (speedup 1.0000x reference)
import jax
import jax.numpy as jnp
from jax.experimental import pallas as pl


def kernel(x, coords, input_dim, W_ft, b_ft, Wb, bb, Wm, bm):
    raise NotImplementedError("write your pallas kernel here")


# speedup vs baseline: 5.4822x; 5.4822x over previous
import functools

import jax
import jax.numpy as jnp
from jax import lax
from jax.experimental import pallas as pl
from jax.experimental.pallas import tpu as pltpu
from jax.experimental.pallas import tpu_sc as plsc

N = 10000
H = 512
K = 5
NLAYERS = 4

NB = 256
NPAD = 10240
NBLK = NPAD // NB
NBK = 128
NBLKK = NPAD // NBK

NW = 32
PW = NPAD // NW
CH = 16
NCH = PW // CH
GROWS = K * CH

_INF = float("inf")


def _knn_body(a_ref, bt_ref, out_ref):
    a = a_ref[...]
    bt = bt_ref[...]
    dot = lax.dot_general(a, bt, (((1,), (0,)), ((), ())),
                          preferred_element_type=jnp.float32)
    sq_i = jnp.sum(a * a, axis=1, keepdims=True)
    sq_j = jnp.sum(bt * bt, axis=0, keepdims=True)
    d2 = (sq_i + sq_j) - 2.0 * dot
    col = lax.broadcasted_iota(jnp.int32, (NBK, NPAD), 1)
    d2 = jnp.where(col < N, d2, _INF)
    for k in range(K):
        m = jnp.min(d2, axis=1, keepdims=True)
        j = jnp.min(jnp.where(d2 == m, col, NPAD), axis=1)
        out_ref[k, :] = j
        d2 = jnp.where(col == j[:, None], _INF, d2)


_knn_call = pl.pallas_call(
    _knn_body,
    grid=(NBLKK,),
    in_specs=[
        pl.BlockSpec((NBK, 8), lambda i: (i, 0)),
        pl.BlockSpec((8, NPAD), lambda i: (0, 0)),
    ],
    out_specs=pl.BlockSpec((8, NBK), lambda i: (0, i)),
    out_shape=jax.ShapeDtypeStruct((8, NPAD), jnp.int32),
)


def _ft_body(x_ref, w_ref, b_ref, out_ref):
    z = lax.dot_general(x_ref[...], w_ref[...], (((0,), (1,)), ((), ())),
                        preferred_element_type=jnp.float32)
    out_ref[...] = jnp.maximum(z + b_ref[...], 0.0)


_ft_call = pl.pallas_call(
    _ft_body,
    grid=(NBLK,),
    in_specs=[
        pl.BlockSpec((H, NB), lambda i: (0, i)),
        pl.BlockSpec((H, H), lambda i: (0, 0)),
        pl.BlockSpec((1, H), lambda i: (0, 0)),
    ],
    out_specs=pl.BlockSpec((NB, H), lambda i: (i, 0)),
    out_shape=jax.ShapeDtypeStruct((NPAD, H), jnp.float32),
)


def _bel_body(agg_ref, w_ref, b_ref, out_ref):
    aggm = agg_ref[...] / 5.0
    z = lax.dot_general(aggm, w_ref[...], (((1,), (1,)), ((), ())),
                        preferred_element_type=jnp.float32)
    out_ref[...] = jnp.maximum(z + b_ref[...], 0.0)


_bel_call = pl.pallas_call(
    _bel_body,
    grid=(NBLK,),
    in_specs=[
        pl.BlockSpec((NB, H), lambda i: (i, 0)),
        pl.BlockSpec((H, H), lambda i: (0, 0)),
        pl.BlockSpec((1, H), lambda i: (0, 0)),
    ],
    out_specs=pl.BlockSpec((NB, H), lambda i: (i, 0)),
    out_shape=jax.ShapeDtypeStruct((NPAD, H), jnp.float32),
)


def _mc_body(agg_ref, h_ref, w_ref, b_ref, out_ref):
    h = h_ref[...]
    curv = agg_ref[...] / 5.0 - h
    z = h + lax.dot_general(curv, w_ref[...], (((1,), (1,)), ((), ())),
                            preferred_element_type=jnp.float32) + b_ref[...]
    out_ref[...] = jnp.maximum(z, 0.0)


_mc_call = pl.pallas_call(
    _mc_body,
    grid=(NBLK,),
    in_specs=[
        pl.BlockSpec((NB, H), lambda i: (i, 0)),
        pl.BlockSpec((NB, H), lambda i: (i, 0)),
        pl.BlockSpec((H, H), lambda i: (0, 0)),
        pl.BlockSpec((1, H), lambda i: (0, 0)),
    ],
    out_specs=pl.BlockSpec((NB, H), lambda i: (i, 0)),
    out_shape=jax.ShapeDtypeStruct((NPAD, H), jnp.float32),
)


def _gather_body(h_hbm, idx_hbm, out_hbm, idx_v, buf_v, acc_v, sem):
    wid = lax.axis_index("s") * 2 + lax.axis_index("c")

    def chunk(c, carry):
        goff = wid * (PW * K) + c * GROWS
        pltpu.sync_copy(idx_hbm.at[pl.ds(goff, GROWS)], idx_v)
        pltpu.async_copy(h_hbm.at[idx_v], buf_v, sem).wait()

        def node(n, carry2):
            def colv(j, carry3):
                s = j * 16
                v = buf_v[n, pl.ds(s, 16)]
                for k in range(1, K):
                    v = v + buf_v[k * CH + n, pl.ds(s, 16)]
                acc_v[n, pl.ds(s, 16)] = v
                return carry3

            return lax.fori_loop(0, H // 16, colv, carry2)

        lax.fori_loop(0, CH, node, 0)
        pltpu.sync_copy(acc_v, out_hbm.at[pl.ds(wid * PW + c * CH, CH)])
        return carry

    lax.fori_loop(0, NCH, chunk, 0)


_gather_call = pl.kernel(
    _gather_body,
    mesh=plsc.VectorSubcoreMesh(core_axis_name="c", subcore_axis_name="s"),
    out_type=jax.ShapeDtypeStruct((NPAD, H), jnp.float32),
    scratch_types=[
        pltpu.VMEM((GROWS,), jnp.int32),
        pltpu.VMEM((GROWS, H), jnp.float32),
        pltpu.VMEM((CH, H), jnp.float32),
        pltpu.SemaphoreType.DMA,
    ],
)


def kernel(x, coords, input_dim, W_ft, b_ft, Wb, bb, Wm, bm):
    pts = coords.T.astype(jnp.float32)
    P = jnp.pad(pts, ((0, NPAD - N), (0, 5)))
    idx8 = _knn_call(P, P.T)
    idx = jnp.clip(idx8[:K], 0, NPAD - 1)
    idxr = (idx.reshape(K, NW, NCH, CH)
               .transpose(1, 2, 0, 3)
               .reshape(-1))

    x0 = jnp.pad(x[0], ((0, 0), (0, NPAD - N)))
    h = _ft_call(x0, W_ft, b_ft.reshape(1, H))
    for l in range(NLAYERS):
        aggs = _gather_call(h, idxr)
        h = _bel_call(aggs, Wb[l], bb[l].reshape(1, H))
    for l in range(NLAYERS):
        aggs = _gather_call(h, idxr)
        h = _mc_call(aggs, h, Wm[l], bm[l].reshape(1, H))
    return h[:N].T.reshape(1, H, N)
